# Initial kernel scaffold; baseline (speedup 1.0000x reference)
import jax
import jax.numpy as jnp
from jax.experimental import pallas as pl


def kernel(x, edge_index, edge_attr, batch, W1, b1, W2, b2, fW1, fb1, fW2, fb2):
    raise NotImplementedError("write your pallas kernel here")


# speedup vs baseline: 15.8000x; 15.8000x over previous
import functools

import jax
import jax.numpy as jnp
from jax import lax
from jax.experimental import pallas as pl
from jax.experimental.pallas import tpu as pltpu
from jax.experimental.pallas import tpu_sc as plsc

N = 10000
E = 320000
G = 64
NC, NS, L = 2, 16, 16
NW = NC * NS
EPT = E // NW
KB = 80
NBLK = EPT // KB
NP = 10240
RPT = NP // NS
NR = 640
DEG_CH = 5

_mesh = plsc.VectorSubcoreMesh(
    core_axis_name="c", subcore_axis_name="s", num_cores=NC, num_subcores=NS)
_sc_params = pltpu.CompilerParams(needs_layout_passes=False)
_sc_params_lin = pltpu.CompilerParams(
    needs_layout_passes=False, use_tc_tiling_on_sc=False)


def _deg_body(dst_hbm, ew_hbm, out_hbm, dstall, ewall, degbuf):
    c = lax.axis_index("c")
    s = lax.axis_index("s")
    wid = c * NS + s

    def zero_row(i, _):
        degbuf[i, :] = jnp.zeros((L,), jnp.float32)
        return 0
    lax.fori_loop(0, NR, zero_row, 0)

    pltpu.sync_copy(dst_hbm.at[pl.ds(wid * EPT, EPT)], dstall)
    pltpu.sync_copy(ew_hbm.at[pl.ds(wid * EPT, EPT)], ewall)

    def acc_body(i, _):
        dv = dstall[pl.ds(i * L, L)]
        wv = ewall[pl.ds(i * L, L)]
        plsc.addupdate_scatter(degbuf, [dv >> 4, dv & 15], wv)
        return 0
    lax.fori_loop(0, EPT // L, acc_body, 0)

    pltpu.sync_copy(degbuf, out_hbm.at[wid])


_deg_kernel = pl.kernel(
    _deg_body,
    out_type=jax.ShapeDtypeStruct((NW, NR, L), jnp.float32),
    mesh=_mesh,
    compiler_params=_sc_params,
    scratch_types=[
        pltpu.VMEM((EPT,), jnp.int32),
        pltpu.VMEM((EPT,), jnp.float32),
        pltpu.VMEM((NR, L), jnp.float32),
    ],
)


def _agg_body(D, t_hbm, src_hbm, dst_hbm, ew_hbm, out_hbm,
              srcall, dstall, ewall, srcbuf, dstbuf, rows, acc, sem):
    c = lax.axis_index("c")
    s = lax.axis_index("s")
    wid = c * NS + s
    FV = D // L

    def zrow(i, _):
        for f in range(FV):
            rows[i, pl.ds(f * L, L)] = jnp.zeros((L,), jnp.float32)
        return 0
    lax.fori_loop(0, KB, zrow, 0)
    for k in range(RPT // KB):
        pltpu.sync_copy(rows, acc.at[pl.ds(s * RPT + k * KB, KB)])

    pltpu.sync_copy(src_hbm.at[pl.ds(wid * EPT, EPT)], srcall)
    pltpu.sync_copy(dst_hbm.at[pl.ds(wid * EPT, EPT)], dstall)
    pltpu.sync_copy(ew_hbm.at[pl.ds(wid * EPT, EPT)], ewall)
    plsc.subcore_barrier()

    def blk_body(j, _):
        eb = j * KB
        for g in range(KB // L):
            srcbuf[pl.ds(g * L, L)] = srcall[pl.ds(eb + g * L, L)]
            dstbuf[pl.ds(g * L, L)] = dstall[pl.ds(eb + g * L, L)]
        pltpu.async_copy(t_hbm.at[srcbuf], rows, sem).wait()

        def grp_body(g, _):
            e0 = eb + g * L
            r0 = g * L
            for i in range(L):
                w = plsc.load_gather(
                    ewall, [jnp.full((L,), e0 + i, jnp.int32)])
                for f in range(FV):
                    rows[r0 + i, pl.ds(f * L, L)] = (
                        rows[r0 + i, pl.ds(f * L, L)] * w)
            return 0
        lax.fori_loop(0, KB // L, grp_body, 0)

        pltpu.sync_copy(rows, acc.at[dstbuf], add=True)
        return 0
    lax.fori_loop(0, NBLK, blk_body, 0)
    plsc.subcore_barrier()

    pltpu.sync_copy(acc.at[pl.ds(s * RPT, RPT)],
                    out_hbm.at[c, pl.ds(s * RPT, RPT)])


def _make_agg_kernel(D):
    return pl.kernel(
        functools.partial(_agg_body, D),
        out_type=jax.ShapeDtypeStruct((NC, NP, D), jnp.float32),
        mesh=_mesh,
        compiler_params=(_sc_params if D == 128 else _sc_params_lin),
        scratch_types=[
            pltpu.VMEM((EPT,), jnp.int32),
            pltpu.VMEM((EPT,), jnp.int32),
            pltpu.VMEM((EPT,), jnp.float32),
            pltpu.VMEM((KB,), jnp.int32),
            pltpu.VMEM((KB,), jnp.int32),
            pltpu.VMEM((KB, D), jnp.float32),
            pltpu.VMEM_SHARED((NP, D), jnp.float32),
            pltpu.SemaphoreType.DMA,
        ],
    )


_agg128 = _make_agg_kernel(128)
_agg64 = _make_agg_kernel(64)


def _tc1_body(degp_ref, x_ref, w1_ref, p1_ref, t1_ref, dis_ref):
    deg = lax.dot_general(degp_ref[...], jnp.ones((NW, 1), jnp.float32),
                          (((0,), (0,)), ((), ())),
                          preferred_element_type=jnp.float32)[:N]
    dis = lax.rsqrt(deg + 1.0)
    p1 = jnp.dot(x_ref[...], w1_ref[...], preferred_element_type=jnp.float32)
    p1_ref[...] = p1
    t1_ref[...] = dis * p1
    dis_ref[...] = dis


def _tc2_body(aggp_ref, p1_ref, dis_ref, b1_ref, w2_ref, p2_ref, t2_ref):
    agg = aggp_ref[0, :N] + aggp_ref[1, :N]
    dis = dis_ref[...]
    h = jnp.maximum(dis * agg + dis * dis * p1_ref[...] + b1_ref[...], 0.0)
    p2 = jnp.dot(h, w2_ref[...], preferred_element_type=jnp.float32)
    p2_ref[...] = p2
    t2_ref[...] = dis * p2


def _tc3_body(aggp_ref, p2_ref, dis_ref, b2_ref, batch_ref,
              fw1_ref, fb1_ref, fw2_ref, fb2_ref, out_ref):
    agg = aggp_ref[0, :N] + aggp_ref[1, :N]
    dis = dis_ref[...]
    x1 = jnp.maximum(dis * agg + dis * dis * p2_ref[...] + b2_ref[...], 0.0)
    gids = lax.broadcasted_iota(jnp.int32, (G, N), 0)
    m = (gids == batch_ref[...]).astype(jnp.float32)
    seg = jnp.dot(m, x1, preferred_element_type=jnp.float32)
    cnt = jnp.sum(m, axis=1, keepdims=True)
    x2 = seg / jnp.maximum(cnt, 1.0)
    hfc = jnp.maximum(
        jnp.dot(x2, fw1_ref[...], preferred_element_type=jnp.float32)
        + fb1_ref[...], 0.0)
    logits = (jnp.dot(hfc, fw2_ref[...], preferred_element_type=jnp.float32)
              + fb2_ref[...])
    mx = jnp.max(logits, axis=1, keepdims=True)
    sh = logits - mx
    out_ref[...] = sh - jnp.log(jnp.sum(jnp.exp(sh), axis=1, keepdims=True))


_tc1 = pl.pallas_call(
    _tc1_body,
    out_shape=(jax.ShapeDtypeStruct((N, 128), jnp.float32),
               jax.ShapeDtypeStruct((N, 128), jnp.float32),
               jax.ShapeDtypeStruct((N, 1), jnp.float32)))

_tc2 = pl.pallas_call(
    _tc2_body,
    out_shape=(jax.ShapeDtypeStruct((N, 64), jnp.float32),
               jax.ShapeDtypeStruct((N, 64), jnp.float32)))

_tc3 = pl.pallas_call(
    _tc3_body,
    out_shape=jax.ShapeDtypeStruct((G, 2), jnp.float32))


def kernel(x, edge_index, edge_attr, batch, W1, b1, W2, b2, fW1, fb1, fW2, fb2):
    src = edge_index[0]
    dst = edge_index[1]
    ew = edge_attr.reshape(E)

    degp = _deg_kernel(dst, ew).reshape(NW, NR * L)

    p1, t1, dis = _tc1(degp, x, W1)
    agg1 = _agg128(t1, src, dst, ew)
    p2, t2 = _tc2(agg1, p1, dis, b1.reshape(1, 128), W2)
    agg2 = _agg64(t2, src, dst, ew)
    return _tc3(agg2, p2, dis, b2.reshape(1, 64), batch.reshape(1, N),
                fW1, fb1.reshape(1, 128), fW2, fb2.reshape(1, 2))
